# baseline (speedup 1.0000x reference)
import functools

import jax
import jax.numpy as jnp
from jax import lax
from jax.experimental import pallas as pl
from jax.experimental.pallas import tpu as pltpu
from jax.experimental.pallas import tpu_sc as plsc

ROWS = 128
COLS = 100000
CHUNK = 50000
CHUNKS_PER_ROW = COLS // CHUNK
NUM_TILES = 32
ROWS_PER_TILE = ROWS // NUM_TILES
LANES = 16
VECS_PER_CHUNK = CHUNK // LANES

_BIG_I32 = 2**31 - 1


def _scan_chunk(buf, base_idx, best_val, best_idx):
    lane = lax.iota(jnp.int32, LANES)

    def body(i, carry):
        bv, bi = carry
        v = buf[pl.ds(i * LANES, LANES)]
        idx = (base_idx + i * LANES) + lane
        m = v > bv
        bv = jnp.where(m, v, bv)
        bi = jnp.where(m, idx, bi)
        return bv, bi

    return lax.fori_loop(0, VECS_PER_CHUNK, body, (best_val, best_idx))


@functools.partial(
    pl.kernel,
    out_type=jax.ShapeDtypeStruct((NUM_TILES * LANES,), jnp.float32),
    mesh=plsc.VectorSubcoreMesh(core_axis_name="c", subcore_axis_name="s"),
    scratch_types=[
        pltpu.VMEM((CHUNK,), jnp.float32),
        pltpu.VMEM((CHUNK,), jnp.float32),
        pltpu.VMEM((LANES,), jnp.float32),
        pltpu.SemaphoreType.DMA,
        pltpu.SemaphoreType.DMA,
    ],
    compiler_params=pltpu.CompilerParams(needs_layout_passes=False),
)
def _argmax_sc(logits_hbm, out_hbm, buf0, buf1, res_v, sem0, sem1):
    wid = lax.axis_index("s") * 2 + lax.axis_index("c")
    row0 = wid * ROWS_PER_TILE
    bufs = (buf0, buf1)
    sems = (sem0, sem1)

    n_chunks = ROWS_PER_TILE * CHUNKS_PER_ROW

    def start(t):
        r = t // CHUNKS_PER_ROW
        c = t % CHUNKS_PER_ROW
        off = (row0 + r) * COLS + c * CHUNK
        return pltpu.async_copy(
            logits_hbm.at[pl.ds(off, CHUNK)],
            bufs[t % 2],
            sems[t % 2],
        )

    copies = [None] * n_chunks
    copies[0] = start(0)

    lane = lax.iota(jnp.int32, LANES)
    res = jnp.zeros((LANES,), jnp.float32)
    best_val = jnp.full((LANES,), -jnp.inf, jnp.float32)
    best_idx = jnp.zeros((LANES,), jnp.int32)
    for t in range(n_chunks):
        if t + 1 < n_chunks:
            copies[t + 1] = start(t + 1)
        copies[t].wait()
        c = t % CHUNKS_PER_ROW
        best_val, best_idx = _scan_chunk(
            bufs[t % 2], jnp.int32(c * CHUNK), best_val, best_idx
        )
        if c == CHUNKS_PER_ROW - 1:
            m = jnp.max(best_val)
            cand = jnp.where(best_val == m, best_idx, jnp.int32(_BIG_I32))
            win = jnp.min(cand).astype(jnp.float32)
            r = t // CHUNKS_PER_ROW
            res = jnp.where(lane == r, win, res)
            best_val = jnp.full((LANES,), -jnp.inf, jnp.float32)
            best_idx = jnp.zeros((LANES,), jnp.int32)

    res_v[...] = res
    pltpu.sync_copy(res_v, out_hbm.at[pl.ds(wid * LANES, LANES)])


def kernel(logits):
    flat = logits.reshape(ROWS * COLS)
    out = _argmax_sc(flat)
    return out.reshape(NUM_TILES, LANES)[:, :ROWS_PER_TILE].reshape(ROWS)


# speedup vs baseline: 1.6249x; 1.6249x over previous
import functools

import jax
import jax.numpy as jnp
from jax import lax
from jax.experimental import pallas as pl
from jax.experimental.pallas import tpu as pltpu
from jax.experimental.pallas import tpu_sc as plsc

ROWS = 128
COLS = 100000
CHUNK = 50000
CHUNKS_PER_ROW = COLS // CHUNK
NUM_TILES = 32
ROWS_PER_TILE = ROWS // NUM_TILES
LANES = 16
VECS_PER_CHUNK = CHUNK // LANES

NACC = 5
GROUPS = 5
BODY = NACC * GROUPS
STEPS = VECS_PER_CHUNK // BODY

_BIG_I32 = 2**31 - 1


def _scan_chunk(buf, chunk_vec_base, accs):
    def body(k, accs):
        accs = list(accs)
        s0 = chunk_vec_base + k * BODY
        for g in range(GROUPS):
            for u in range(NACC):
                j = g * NACC + u
                v = buf[pl.ds((k * BODY + j) * LANES, LANES)]
                s = jnp.broadcast_to(s0 + j, (LANES,))
                bv, bs = accs[u]
                m = v > bv
                accs[u] = (jnp.where(m, v, bv), jnp.where(m, s, bs))
        return tuple(accs)

    return lax.fori_loop(0, STEPS, body, tuple(accs))


@functools.partial(
    pl.kernel,
    out_type=jax.ShapeDtypeStruct((NUM_TILES * LANES,), jnp.float32),
    mesh=plsc.VectorSubcoreMesh(core_axis_name="c", subcore_axis_name="s"),
    scratch_types=[
        pltpu.VMEM((CHUNK,), jnp.float32),
        pltpu.VMEM((CHUNK,), jnp.float32),
        pltpu.VMEM((LANES,), jnp.float32),
        pltpu.SemaphoreType.DMA,
        pltpu.SemaphoreType.DMA,
    ],
    compiler_params=pltpu.CompilerParams(needs_layout_passes=False),
)
def _argmax_sc(logits_hbm, out_hbm, buf0, buf1, res_v, sem0, sem1):
    wid = lax.axis_index("s") * 2 + lax.axis_index("c")
    row0 = wid * ROWS_PER_TILE
    bufs = (buf0, buf1)
    sems = (sem0, sem1)

    n_chunks = ROWS_PER_TILE * CHUNKS_PER_ROW

    def start(t):
        r = t // CHUNKS_PER_ROW
        c = t % CHUNKS_PER_ROW
        off = (row0 + r) * COLS + c * CHUNK
        return pltpu.async_copy(
            logits_hbm.at[pl.ds(off, CHUNK)],
            bufs[t % 2],
            sems[t % 2],
        )

    def fresh_accs():
        return [
            (
                jnp.full((LANES,), -jnp.inf, jnp.float32),
                jnp.zeros((LANES,), jnp.int32),
            )
            for _ in range(NACC)
        ]

    copies = [None] * n_chunks
    copies[0] = start(0)

    lane = lax.iota(jnp.int32, LANES)
    res = jnp.zeros((LANES,), jnp.float32)
    accs = fresh_accs()
    for t in range(n_chunks):
        if t + 1 < n_chunks:
            copies[t + 1] = start(t + 1)
        copies[t].wait()
        c = t % CHUNKS_PER_ROW
        accs = _scan_chunk(bufs[t % 2], jnp.int32(c * VECS_PER_CHUNK), accs)
        if c == CHUNKS_PER_ROW - 1:
            bv, bi = accs[0][0], accs[0][1] * LANES + lane
            for u in range(1, NACC):
                v2, i2 = accs[u][0], accs[u][1] * LANES + lane
                better = (v2 > bv) | ((v2 == bv) & (i2 < bi))
                bv = jnp.where(better, v2, bv)
                bi = jnp.where(better, i2, bi)
            m = jnp.max(bv)
            cand = jnp.where(bv == m, bi, jnp.int32(_BIG_I32))
            win = jnp.min(cand).astype(jnp.float32)
            r = t // CHUNKS_PER_ROW
            res = jnp.where(lane == r, win, res)
            accs = fresh_accs()

    res_v[...] = res
    pltpu.sync_copy(res_v, out_hbm.at[pl.ds(wid * LANES, LANES)])


def kernel(logits):
    flat = logits.reshape(ROWS * COLS)
    out = _argmax_sc(flat)
    return out.reshape(NUM_TILES, LANES)[:, :ROWS_PER_TILE].reshape(ROWS)
